# baseline (speedup 1.0000x reference)
import functools

import jax
import jax.numpy as jnp
from jax import lax
from jax.experimental import pallas as pl
from jax.experimental.pallas import tpu as pltpu
from jax.experimental.pallas import tpu_sc as plsc

N = 10000
E = 160000
DIN, H1, H2, DOUT = 256, 512, 512, 256

EPAD = 163840
PER_TILE = EPAD // 16
NB = PER_TILE // 128
ACC_ROWS = 10240
DUMMY_ROW = N
BLK = 1000


def _deg_body(dst_hbm, ones_hbm, zeros_hbm, out_hbm, dst_v, ones_v, acc, sem):
    c = lax.axis_index("c")
    s = lax.axis_index("s")
    @pl.when(c == 0)
    def _():
        pltpu.sync_copy(dst_hbm.at[s], dst_v)
        pltpu.sync_copy(ones_hbm, ones_v)
        pltpu.sync_copy(zeros_hbm, acc.at[pl.ds(s * 640, 640)])
        plsc.subcore_barrier()
        def body(j, carry):
            pltpu.sync_copy(ones_v, acc.at[dst_v.at[j]], add=True)
            return carry
        lax.fori_loop(0, NB, body, 0)
        plsc.subcore_barrier()
        @pl.when(s < 15)
        def _():
            pltpu.sync_copy(acc.at[pl.ds(s * 640, 640)],
                            out_hbm.at[pl.ds(s * 640, 640)])
        @pl.when(s == 15)
        def _():
            pltpu.sync_copy(acc.at[pl.ds(9600, 400)],
                            out_hbm.at[pl.ds(9600, 400)])


_deg_call = functools.partial(
    pl.kernel,
    out_type=jax.ShapeDtypeStruct((N, 128), jnp.float32),
    mesh=plsc.VectorSubcoreMesh(core_axis_name="c", subcore_axis_name="s"),
    scratch_types=[
        pltpu.VMEM((NB, 128), jnp.int32),
        pltpu.VMEM((128, 128), jnp.float32),
        pltpu.VMEM_SHARED((ACC_ROWS, 128), jnp.float32),
        pltpu.SemaphoreType.DMA,
    ],
)(_deg_body)


def _make_agg(n_chunks):
    cpc = n_chunks // 2

    HB = NB // 2

    def body(src_hbm, dst_hbm, zeros_hbm, *rest):
        g_refs = rest[:n_chunks]
        out_refs = rest[n_chunks:2 * n_chunks]
        src_v, dst_v, rows0, rows1, acc, sem0, sem1 = rest[2 * n_chunks:]
        c = lax.axis_index("c")
        s = lax.axis_index("s")
        for ci in range(n_chunks):
            @pl.when(c == ci // cpc)
            def _(g_ref=g_refs[ci], out_ref=out_refs[ci]):
                pltpu.sync_copy(zeros_hbm, acc.at[pl.ds(s * 640, 640)])
                plsc.subcore_barrier()
                for h in range(2):
                    pltpu.sync_copy(src_hbm.at[s, pl.ds(h * HB, HB)], src_v)
                    pltpu.sync_copy(dst_hbm.at[s, pl.ds(h * HB, HB)], dst_v)
                    pltpu.async_copy(g_ref.at[src_v.at[0]], rows0, sem0)
                    def pair(i, carry):
                        pltpu.make_async_copy(
                            g_ref.at[src_v.at[2 * i]], rows0, sem0).wait()
                        pltpu.async_copy(
                            g_ref.at[src_v.at[2 * i + 1]], rows1, sem1)
                        pltpu.sync_copy(rows0, acc.at[dst_v.at[2 * i]],
                                        add=True)
                        @pl.when(i < HB // 2 - 1)
                        def _():
                            pltpu.async_copy(
                                g_ref.at[src_v.at[2 * i + 2]], rows0, sem0)
                        pltpu.make_async_copy(
                            g_ref.at[src_v.at[2 * i + 1]], rows1, sem1).wait()
                        pltpu.sync_copy(rows1, acc.at[dst_v.at[2 * i + 1]],
                                        add=True)
                        return carry
                    lax.fori_loop(0, HB // 2, pair, 0)
                plsc.subcore_barrier()
                @pl.when(s < 15)
                def _():
                    pltpu.sync_copy(acc.at[pl.ds(s * 640, 640)],
                                    out_ref.at[pl.ds(s * 640, 640)])
                @pl.when(s == 15)
                def _():
                    pltpu.sync_copy(acc.at[pl.ds(9600, 400)],
                                    out_ref.at[pl.ds(9600, 400)])
                plsc.subcore_barrier()

    return functools.partial(
        pl.kernel,
        out_type=[jax.ShapeDtypeStruct((N, 128), jnp.float32)] * n_chunks,
        mesh=plsc.VectorSubcoreMesh(core_axis_name="c", subcore_axis_name="s"),
        scratch_types=[
            pltpu.VMEM((HB, 128), jnp.int32),
            pltpu.VMEM((HB, 128), jnp.int32),
            pltpu.VMEM((128, 128), jnp.float32),
            pltpu.VMEM((128, 128), jnp.float32),
            pltpu.VMEM_SHARED((ACC_ROWS, 128), jnp.float32),
            pltpu.SemaphoreType.DMA,
            pltpu.SemaphoreType.DMA,
        ],
    )(body)


_agg4 = _make_agg(4)
_agg2 = _make_agg(2)


def _dinv_of(deg_ref):
    return lax.rsqrt(deg_ref[:, 0:1] + 1.0)


def _store_chunks(outs, y):
    for i, o in enumerate(outs):
        o[...] = y[:, i * 128:(i + 1) * 128]


def _mm1_body(x_ref, w_ref, deg_ref, *outs):
    dinv = _dinv_of(deg_ref)
    g = jnp.dot(x_ref[...], w_ref[...],
                preferred_element_type=jnp.float32) * dinv
    _store_chunks(outs, g)


def _mm_mid_body(b_ref, w_ref, deg_ref, *rest):
    nc_in = 4
    a_refs, g_refs = rest[:nc_in], rest[nc_in:2 * nc_in]
    outs = rest[2 * nc_in:]
    dinv = _dinv_of(deg_ref)
    h = jnp.concatenate(
        [a_refs[i][...] + g_refs[i][...] for i in range(nc_in)], axis=1)
    h = jnp.maximum(h * dinv + b_ref[...], 0.0)
    y = jnp.dot(h, w_ref[...], preferred_element_type=jnp.float32) * dinv
    _store_chunks(outs, y)


def _ep_body(b_ref, deg_ref, a0, a1, g0, g1, out):
    dinv = _dinv_of(deg_ref)
    h = jnp.concatenate([a0[...] + g0[...], a1[...] + g1[...]], axis=1)
    out[...] = h * dinv + b_ref[...]


def _row_spec(cols):
    return pl.BlockSpec((BLK, cols), lambda i: (i, 0))


_DEG_SPEC = pl.BlockSpec((BLK, 128), lambda i: (i, 0))


def _mm_out(nco):
    return dict(
        out_specs=[_row_spec(128)] * nco,
        out_shape=[jax.ShapeDtypeStruct((N, 128), jnp.float32)] * nco,
    )


def _mm1_call(x, w, deg16):
    return pl.pallas_call(
        _mm1_body,
        grid=(N // BLK,),
        in_specs=[_row_spec(x.shape[1]),
                  pl.BlockSpec(w.shape, lambda i: (0, 0)),
                  _DEG_SPEC],
        **_mm_out(w.shape[1] // 128),
    )(x, w, deg16)


def _mm_mid_call(aggs, gs, b, w, deg16):
    return pl.pallas_call(
        _mm_mid_body,
        grid=(N // BLK,),
        in_specs=[pl.BlockSpec((1, w.shape[0]), lambda i: (0, 0)),
                  pl.BlockSpec(w.shape, lambda i: (0, 0)),
                  _DEG_SPEC] + [_row_spec(128)] * 8,
        **_mm_out(w.shape[1] // 128),
    )(b.reshape(1, -1), w, deg16, *aggs, *gs)


def _ep_call(aggs, gs, b, deg16):
    return pl.pallas_call(
        _ep_body,
        grid=(N // BLK,),
        in_specs=[pl.BlockSpec((1, DOUT), lambda i: (0, 0)), _DEG_SPEC]
                 + [_row_spec(128)] * 4,
        out_specs=_row_spec(DOUT),
        out_shape=jax.ShapeDtypeStruct((N, DOUT), jnp.float32),
    )(b.reshape(1, -1), deg16, *aggs, *gs)


def kernel(x, edge_index, W1, b1, W2, b2, W3, b3):
    src = edge_index[0].astype(jnp.int32)
    dst = edge_index[1].astype(jnp.int32)
    pad = EPAD - E
    src_p = jnp.concatenate(
        [src, jnp.zeros((pad,), jnp.int32)]).reshape(16, NB, 128)
    dst_p = jnp.concatenate(
        [dst, jnp.full((pad,), DUMMY_ROW, jnp.int32)]).reshape(16, NB, 128)
    zeros128 = jnp.zeros((640, 128), jnp.float32)
    ones128 = jnp.ones((128, 128), jnp.float32)

    deg16 = _deg_call(dst_p, ones128, zeros128)
    g1 = _mm1_call(x, W1, deg16)
    a1 = _agg4(src_p, dst_p, zeros128, *g1)
    g2 = _mm_mid_call(a1, g1, b1, W2, deg16)
    a2 = _agg4(src_p, dst_p, zeros128, *g2)
    g3 = _mm_mid_call(a2, g2, b2, W3, deg16)
    a3 = _agg2(src_p, dst_p, zeros128, *g3)
    return _ep_call(a3, g3, b3, deg16)


# speedup vs baseline: 1.0461x; 1.0461x over previous
import functools

import jax
import jax.numpy as jnp
from jax import lax
from jax.experimental import pallas as pl
from jax.experimental.pallas import tpu as pltpu
from jax.experimental.pallas import tpu_sc as plsc

N = 10000
E = 160000
DIN, H1, H2, DOUT = 256, 512, 512, 256

EPAD = 163840
PER_TILE = EPAD // 16
NB = PER_TILE // 128
ACC_ROWS = 10240
DUMMY_ROW = N
BLK = 1000


def _deg_body(dst_hbm, ones_hbm, zeros_hbm, out_hbm, dst_v, ones_v, acc, sem):
    c = lax.axis_index("c")
    s = lax.axis_index("s")
    @pl.when(c == 0)
    def _():
        pltpu.sync_copy(dst_hbm.at[s], dst_v)
        pltpu.sync_copy(ones_hbm, ones_v)
        pltpu.sync_copy(zeros_hbm, acc.at[pl.ds(s * 640, 640)])
        plsc.subcore_barrier()
        def body(j, carry):
            pltpu.sync_copy(ones_v, acc.at[dst_v.at[j]], add=True)
            return carry
        lax.fori_loop(0, NB, body, 0)
        plsc.subcore_barrier()
        @pl.when(s < 15)
        def _():
            pltpu.sync_copy(acc.at[pl.ds(s * 640, 640)],
                            out_hbm.at[pl.ds(s * 640, 640)])
        @pl.when(s == 15)
        def _():
            pltpu.sync_copy(acc.at[pl.ds(9600, 400)],
                            out_hbm.at[pl.ds(9600, 400)])


_deg_call = functools.partial(
    pl.kernel,
    out_type=jax.ShapeDtypeStruct((N, 128), jnp.float32),
    mesh=plsc.VectorSubcoreMesh(core_axis_name="c", subcore_axis_name="s"),
    scratch_types=[
        pltpu.VMEM((NB, 128), jnp.int32),
        pltpu.VMEM((128, 128), jnp.float32),
        pltpu.VMEM_SHARED((ACC_ROWS, 128), jnp.float32),
        pltpu.SemaphoreType.DMA,
    ],
)(_deg_body)


def _make_agg(n_chunks):
    cpc = n_chunks // 2

    HB = NB // 2

    def body(src_hbm, dst_hbm, zeros_hbm, *rest):
        g_refs = rest[:n_chunks]
        out_refs = rest[n_chunks:2 * n_chunks]
        src_v, dst_v, rows0, rows1, acc, sem0, sem1 = rest[2 * n_chunks:]
        c = lax.axis_index("c")
        s = lax.axis_index("s")
        for ci in range(n_chunks):
            @pl.when(c == ci // cpc)
            def _(g_ref=g_refs[ci], out_ref=out_refs[ci]):
                pltpu.sync_copy(zeros_hbm, acc.at[pl.ds(s * 640, 640)])
                plsc.subcore_barrier()
                for h in range(2):
                    pltpu.sync_copy(src_hbm.at[s, pl.ds(h * HB, HB)], src_v)
                    pltpu.sync_copy(dst_hbm.at[s, pl.ds(h * HB, HB)], dst_v)
                    pltpu.async_copy(g_ref.at[src_v.at[0]], rows0, sem0)
                    def pair(i, carry):
                        pltpu.make_async_copy(
                            g_ref.at[src_v.at[2 * i]], rows0, sem0).wait()
                        pltpu.async_copy(
                            g_ref.at[src_v.at[2 * i + 1]], rows1, sem1)
                        pltpu.sync_copy(rows0, acc.at[dst_v.at[2 * i]],
                                        add=True)
                        @pl.when(i < HB // 2 - 1)
                        def _():
                            pltpu.async_copy(
                                g_ref.at[src_v.at[2 * i + 2]], rows0, sem0)
                        pltpu.make_async_copy(
                            g_ref.at[src_v.at[2 * i + 1]], rows1, sem1).wait()
                        pltpu.sync_copy(rows1, acc.at[dst_v.at[2 * i + 1]],
                                        add=True)
                        return carry
                    lax.fori_loop(0, HB // 2, pair, 0)
                plsc.subcore_barrier()
                @pl.when(s < 15)
                def _():
                    pltpu.sync_copy(acc.at[pl.ds(s * 640, 640)],
                                    out_ref.at[pl.ds(s * 640, 640)])
                @pl.when(s == 15)
                def _():
                    pltpu.sync_copy(acc.at[pl.ds(9600, 400)],
                                    out_ref.at[pl.ds(9600, 400)])
                plsc.subcore_barrier()

    return functools.partial(
        pl.kernel,
        out_type=[jax.ShapeDtypeStruct((N, 128), jnp.float32)] * n_chunks,
        mesh=plsc.VectorSubcoreMesh(core_axis_name="c", subcore_axis_name="s"),
        scratch_types=[
            pltpu.VMEM((HB, 128), jnp.int32),
            pltpu.VMEM((HB, 128), jnp.int32),
            pltpu.VMEM((128, 128), jnp.float32),
            pltpu.VMEM((128, 128), jnp.float32),
            pltpu.VMEM_SHARED((ACC_ROWS, 128), jnp.float32),
            pltpu.SemaphoreType.DMA,
            pltpu.SemaphoreType.DMA,
        ],
    )(body)


_agg4 = _make_agg(4)
_agg2 = _make_agg(2)


def _dinv_of(deg_ref):
    return lax.rsqrt(deg_ref[:, 0:1] + 1.0)


def _store_chunks(outs, y):
    for i, o in enumerate(outs):
        o[...] = y[:, i * 128:(i + 1) * 128]


def _mm1a_body(x_ref, w_ref, *outs):
    h = jnp.dot(x_ref[...], w_ref[...], preferred_element_type=jnp.float32)
    _store_chunks(outs, h)


def _mm1b_body(deg_ref, *rest):
    h_refs, outs = rest[:4], rest[4:]
    dinv = _dinv_of(deg_ref)
    for h, o in zip(h_refs, outs):
        o[...] = h[...] * dinv


def _mm_mid_body(b_ref, w_ref, deg_ref, *rest):
    nc_in = 4
    a_refs, g_refs = rest[:nc_in], rest[nc_in:2 * nc_in]
    outs = rest[2 * nc_in:]
    dinv = _dinv_of(deg_ref)
    h = jnp.concatenate(
        [a_refs[i][...] + g_refs[i][...] for i in range(nc_in)], axis=1)
    h = jnp.maximum(h * dinv + b_ref[...], 0.0)
    y = jnp.dot(h, w_ref[...], preferred_element_type=jnp.float32) * dinv
    _store_chunks(outs, y)


def _ep_body(b_ref, deg_ref, a0, a1, g0, g1, out):
    dinv = _dinv_of(deg_ref)
    h = jnp.concatenate([a0[...] + g0[...], a1[...] + g1[...]], axis=1)
    out[...] = h * dinv + b_ref[...]


def _row_spec(cols):
    return pl.BlockSpec((BLK, cols), lambda i: (i, 0))


_DEG_SPEC = pl.BlockSpec((BLK, 128), lambda i: (i, 0))


def _mm_out(nco):
    return dict(
        out_specs=[_row_spec(128)] * nco,
        out_shape=[jax.ShapeDtypeStruct((N, 128), jnp.float32)] * nco,
    )


def _mm1_call(x, w, deg16):
    h = pl.pallas_call(
        _mm1a_body,
        grid=(N // BLK,),
        in_specs=[_row_spec(x.shape[1]),
                  pl.BlockSpec(w.shape, lambda i: (0, 0))],
        **_mm_out(w.shape[1] // 128),
    )(x, w)
    return pl.pallas_call(
        _mm1b_body,
        grid=(N // BLK,),
        in_specs=[_DEG_SPEC] + [_row_spec(128)] * 4,
        **_mm_out(4),
    )(deg16, *h)


def _mm_mid_call(aggs, gs, b, w, deg16):
    return pl.pallas_call(
        _mm_mid_body,
        grid=(N // BLK,),
        in_specs=[pl.BlockSpec((1, w.shape[0]), lambda i: (0, 0)),
                  pl.BlockSpec(w.shape, lambda i: (0, 0)),
                  _DEG_SPEC] + [_row_spec(128)] * 8,
        **_mm_out(w.shape[1] // 128),
    )(b.reshape(1, -1), w, deg16, *aggs, *gs)


def _ep_call(aggs, gs, b, deg16):
    return pl.pallas_call(
        _ep_body,
        grid=(N // BLK,),
        in_specs=[pl.BlockSpec((1, DOUT), lambda i: (0, 0)), _DEG_SPEC]
                 + [_row_spec(128)] * 4,
        out_specs=_row_spec(DOUT),
        out_shape=jax.ShapeDtypeStruct((N, DOUT), jnp.float32),
    )(b.reshape(1, -1), deg16, *aggs, *gs)


def kernel(x, edge_index, W1, b1, W2, b2, W3, b3):
    src = edge_index[0].astype(jnp.int32)
    dst = edge_index[1].astype(jnp.int32)
    pad = EPAD - E
    src_p = jnp.concatenate(
        [src, jnp.zeros((pad,), jnp.int32)]).reshape(16, NB, 128)
    dst_p = jnp.concatenate(
        [dst, jnp.full((pad,), DUMMY_ROW, jnp.int32)]).reshape(16, NB, 128)
    zeros128 = jnp.zeros((640, 128), jnp.float32)
    ones128 = jnp.ones((128, 128), jnp.float32)

    deg16 = _deg_call(dst_p, ones128, zeros128)
    g1 = _mm1_call(x, W1, deg16)
    a1 = _agg4(src_p, dst_p, zeros128, *g1)
    g2 = _mm_mid_call(a1, g1, b1, W2, deg16)
    a2 = _agg4(src_p, dst_p, zeros128, *g2)
    g3 = _mm_mid_call(a2, g2, b2, W3, deg16)
    a3 = _agg2(src_p, dst_p, zeros128, *g3)
    return _ep_call(a3, g3, b3, deg16)
